# baseline (device time: 28667 ns/iter reference)
import jax
import jax.numpy as jnp
from jax import lax
from jax.experimental import pallas as pl
from jax.experimental.pallas import tpu as pltpu

T = 512
D = 512
F = 1024
E_LOC = 2
NCHUNK = 4


def kernel(x, assign, W1, W2):
    assign2d = assign.reshape(T, 1)

    def body(x_ref, a_ref, w1_ref, w2_ref, out_ref,
             xb, w1b, w2b, xrecv, arecv, accrem, partner,
             send_sems, recv_sems, ret_send_sems, ret_recv_sems):
        my_x = lax.axis_index("x")
        my_y = lax.axis_index("y")
        nbr = (my_x, 1 - my_y)

        barrier_sem = pltpu.get_barrier_semaphore()
        pl.semaphore_signal(barrier_sem, inc=1, device_id=nbr,
                            device_id_type=pl.DeviceIdType.MESH)
        pl.semaphore_wait(barrier_sem, 1)

        xb[:, :] = x_ref[:, :].astype(jnp.bfloat16)
        rdma_x = pltpu.make_async_remote_copy(
            src_ref=xb, dst_ref=xrecv,
            send_sem=send_sems.at[0], recv_sem=recv_sems.at[0],
            device_id=nbr, device_id_type=pl.DeviceIdType.MESH,
        )
        rdma_a = pltpu.make_async_remote_copy(
            src_ref=a_ref, dst_ref=arecv,
            send_sem=send_sems.at[1], recv_sem=recv_sems.at[1],
            device_id=nbr, device_id_type=pl.DeviceIdType.MESH,
        )
        rdma_x.start()
        rdma_a.start()

        for k in range(E_LOC):
            w1b[k] = w1_ref[k].astype(jnp.bfloat16)
            w2b[k] = w2_ref[k].astype(jnp.bfloat16)

        def moe(tokens, assigns):
            acc = jnp.zeros((tokens.shape[0], D), jnp.float32)
            for k in range(E_LOC):
                e = E_LOC * my_y + k
                h = jnp.maximum(
                    jnp.dot(tokens, w1b[k], preferred_element_type=jnp.float32),
                    0.0).astype(jnp.bfloat16)
                y = jnp.dot(h, w2b[k], preferred_element_type=jnp.float32)
                acc = acc + jnp.where(assigns == e, y, 0.0)
            return acc

        acc_local = moe(xb[:, :], a_ref[:, :])

        rdma_x.wait()
        rdma_a.wait()

        CH = T // NCHUNK
        rets = []
        for c in range(NCHUNK):
            rows = slice(c * CH, (c + 1) * CH)
            accrem[rows, :] = moe(xrecv[rows, :], arecv[rows, :]).astype(jnp.bfloat16)
            r = pltpu.make_async_remote_copy(
                src_ref=accrem.at[rows],
                dst_ref=partner.at[rows],
                send_sem=ret_send_sems.at[c], recv_sem=ret_recv_sems.at[c],
                device_id=nbr, device_id_type=pl.DeviceIdType.MESH,
            )
            r.start()
            rets.append(r)

        for c, r in enumerate(rets):
            rows = slice(c * CH, (c + 1) * CH)
            r.wait_recv()
            out_ref[rows, :] = acc_local[rows, :] + partner[rows, :].astype(jnp.float32)
        for r in rets:
            r.wait_send()

    return pl.pallas_call(
        body,
        out_shape=jax.ShapeDtypeStruct((T, D), jnp.float32),
        in_specs=[
            pl.BlockSpec(memory_space=pltpu.VMEM),
            pl.BlockSpec(memory_space=pltpu.VMEM),
            pl.BlockSpec(memory_space=pltpu.VMEM),
            pl.BlockSpec(memory_space=pltpu.VMEM),
        ],
        out_specs=pl.BlockSpec(memory_space=pltpu.VMEM),
        scratch_shapes=[
            pltpu.VMEM((T, D), jnp.bfloat16),
            pltpu.VMEM((E_LOC, D, F), jnp.bfloat16),
            pltpu.VMEM((E_LOC, F, D), jnp.bfloat16),
            pltpu.VMEM((T, D), jnp.bfloat16),
            pltpu.VMEM((T, 1), jnp.int32),
            pltpu.VMEM((T, D), jnp.bfloat16),
            pltpu.VMEM((T, D), jnp.bfloat16),
            pltpu.SemaphoreType.DMA((2,)),
            pltpu.SemaphoreType.DMA((2,)),
            pltpu.SemaphoreType.DMA((NCHUNK,)),
            pltpu.SemaphoreType.DMA((NCHUNK,)),
        ],
        compiler_params=pltpu.CompilerParams(collective_id=0),
    )(x, assign2d, W1, W2)
